# baseline (speedup 1.0000x reference)
import functools

import jax
import jax.numpy as jnp
from jax import lax
from jax.experimental import pallas as pl
from jax.experimental.pallas import tpu as pltpu
from jax.experimental.pallas import tpu_sc as plsc

NC = 2
NS = 16
NW = NC * NS
LANES = 16
K = 80
ZR = 40


def _sc_mesh():
    return plsc.VectorSubcoreMesh(
        core_axis_name="c", subcore_axis_name="s", num_cores=NC, num_subcores=NS
    )


def _phase1_scatter(edge_attr, src, n_nodes):
    E, D = edge_attr.shape
    N = n_nodes
    EPW = E // NW
    NCH = EPW // K
    G = D // LANES
    NZC = N // ZR
    NBUF = 2

    @functools.partial(
        pl.kernel,
        out_type=(
            jax.ShapeDtypeStruct((NC, N, D), jnp.float32),
            jax.ShapeDtypeStruct((NW, N), jnp.float32),
        ),
        mesh=_sc_mesh(),
        compiler_params=pltpu.CompilerParams(needs_layout_passes=False),
        scratch_types=[
            pltpu.VMEM_SHARED((N, D), jnp.float32),
            [pltpu.VMEM((K, D), jnp.float32) for _ in range(NBUF)],
            pltpu.VMEM((NCH, K), jnp.int32),
            pltpu.VMEM((N,), jnp.float32),
            [pltpu.SemaphoreType.DMA for _ in range(NBUF)],
            [pltpu.SemaphoreType.DMA for _ in range(NBUF)],
        ],
    )
    def scatter_k(edge_hbm, src3_hbm, psum_hbm, pcnt_hbm,
                  acc_sh, rows_v, idx_all, hist_v, sr, ss):
        cid = lax.axis_index("c")
        sid = lax.axis_index("s")
        wid = cid * NS + sid

        pltpu.sync_copy(src3_hbm.at[wid], idx_all)

        @pl.loop(0, ZR)
        def _(r):
            @pl.loop(0, G)
            def _(g):
                rows_v[0][r, pl.ds(g * LANES, LANES)] = jnp.zeros((LANES,), jnp.float32)

        @pl.loop(0, N // LANES)
        def _(g):
            hist_v[pl.ds(g * LANES, LANES)] = jnp.zeros((LANES,), jnp.float32)

        @pl.loop(0, NZC)
        def _(t):
            @pl.when(lax.rem(t, NS) == sid)
            def _():
                pltpu.sync_copy(rows_v[0].at[pl.ds(0, ZR)], acc_sh.at[pl.ds(t * ZR, ZR)])

        plsc.subcore_barrier()

        e0 = wid * EPW
        ones = jnp.ones((LANES,), jnp.float32)

        def fetch(c, j):
            pltpu.async_copy(edge_hbm.at[pl.ds(e0 + c * K, K)], rows_v[j], sr[j])

        def drain_scatter(j):
            pltpu.make_async_copy(rows_v[j], acc_sh.at[idx_all.at[0]], ss[j]).wait()

        def consume(c, j):
            pltpu.make_async_copy(
                edge_hbm.at[pl.ds(e0, K)], rows_v[j], sr[j]).wait()
            pltpu.async_copy(rows_v[j], acc_sh.at[idx_all.at[c]], ss[j], add=True)
            for g in range(K // LANES):
                iv = idx_all[c, pl.ds(g * LANES, LANES)]
                plsc.addupdate_scatter(hist_v, [iv], ones)

        fetch(0, 0)

        @pl.loop(0, NCH - 1)
        def _(c):
            jn = lax.rem(c + 1, NBUF)
            for j in range(NBUF):
                @pl.when(jn == j)
                def _():
                    @pl.when(c >= NBUF - 1)
                    def _():
                        drain_scatter(j)
                    fetch(c + 1, j)
            jc = lax.rem(c, NBUF)
            for j in range(NBUF):
                @pl.when(jc == j)
                def _():
                    consume(c, j)

        consume(NCH - 1, (NCH - 1) % NBUF)
        for j in range(NBUF):
            drain_scatter(j)

        pltpu.sync_copy(hist_v, pcnt_hbm.at[wid])
        plsc.subcore_barrier()

        @pl.loop(0, NZC)
        def _(t):
            @pl.when(lax.rem(t, NS) == sid)
            def _():
                r0 = t * ZR
                pltpu.sync_copy(acc_sh.at[pl.ds(r0, ZR)], psum_hbm.at[cid, pl.ds(r0, ZR)])

    return scatter_k(edge_attr, src.reshape(NW, NCH, K))


def _phase2_node_update(psum, pcnt_t, node_attr, wt, b2):
    N, D = node_attr.shape
    BN = 1000

    def body(ps_ref, pc_ref, na_ref, wt_ref, b_ref, out_ref):
        counts = jnp.sum(pc_ref[...], axis=1, keepdims=True)
        sums = ps_ref[0, :, :] + ps_ref[1, :, :]
        mean = sums / jnp.maximum(counts, 1.0)
        x = na_ref[...] + 0.5 * mean
        z = jnp.dot(x, wt_ref[...], preferred_element_type=jnp.float32) + b_ref[...]
        out_ref[...] = 0.5 / (1.0 + jnp.exp(-z))

    return pl.pallas_call(
        body,
        grid=(N // BN,),
        in_specs=[
            pl.BlockSpec((NC, BN, D), lambda i: (0, i, 0)),
            pl.BlockSpec((BN, NW), lambda i: (i, 0)),
            pl.BlockSpec((BN, D), lambda i: (i, 0)),
            pl.BlockSpec((D, D), lambda i: (0, 0)),
            pl.BlockSpec((1, D), lambda i: (0, 0)),
        ],
        out_specs=pl.BlockSpec((BN, D), lambda i: (i, 0)),
        out_shape=jax.ShapeDtypeStruct((N, D), jnp.float32),
    )(psum, pcnt_t, node_attr, wt, b2)


def _phase3_gather(emb_half, src, dst):
    N, D = emb_half.shape
    E = src.shape[0]
    EPW = E // NW
    NCH = EPW // K
    G = D // LANES

    NBUF = 3

    @functools.partial(
        pl.kernel,
        out_type=jax.ShapeDtypeStruct((E, D), jnp.float32),
        mesh=_sc_mesh(),
        scratch_types=[
            pltpu.VMEM((EPW,), jnp.int32),
            pltpu.VMEM((EPW,), jnp.int32),
            [pltpu.VMEM((K, D), jnp.float32) for _ in range(NBUF)],
            [pltpu.VMEM((K, D), jnp.float32) for _ in range(NBUF)],
            [pltpu.VMEM((K, D), jnp.float32) for _ in range(NBUF)],
            [pltpu.SemaphoreType.DMA for _ in range(NBUF)],
            [pltpu.SemaphoreType.DMA for _ in range(NBUF)],
            [pltpu.SemaphoreType.DMA for _ in range(NBUF)],
        ],
    )
    def gather_k(emb_hbm, src_hbm, dst_hbm, out_hbm,
                 ia_v, ib_v, ba_v, bb_v, bo_v, sa, sb, so):
        cid = lax.axis_index("c")
        sid = lax.axis_index("s")
        wid = cid * NS + sid
        e0 = wid * EPW

        pltpu.sync_copy(src_hbm.at[pl.ds(e0, EPW)], ia_v)
        pltpu.sync_copy(dst_hbm.at[pl.ds(e0, EPW)], ib_v)

        def fetch(c, j):
            o = c * K
            pltpu.async_copy(emb_hbm.at[ia_v.at[pl.ds(o, K)]], ba_v[j], sa[j])
            pltpu.async_copy(emb_hbm.at[ib_v.at[pl.ds(o, K)]], bb_v[j], sb[j])

        def drain_out(j):
            pltpu.make_async_copy(bo_v[j], out_hbm.at[pl.ds(e0, K)], so[j]).wait()

        def consume(c, j):
            o = c * K
            pltpu.make_async_copy(
                emb_hbm.at[ia_v.at[pl.ds(o, K)]], ba_v[j], sa[j]).wait()
            pltpu.make_async_copy(
                emb_hbm.at[ib_v.at[pl.ds(o, K)]], bb_v[j], sb[j]).wait()

            @plsc.parallel_loop(0, K * G, unroll=8)
            def _(i):
                r = lax.shift_right_logical(i, 3)
                g = lax.shift_left(lax.bitwise_and(i, G - 1), 4)
                bo_v[j][r, pl.ds(g, LANES)] = (
                    ba_v[j][r, pl.ds(g, LANES)] + bb_v[j][r, pl.ds(g, LANES)]
                )

            pltpu.async_copy(bo_v[j], out_hbm.at[pl.ds(e0 + c * K, K)], so[j])

        fetch(0, 0)

        @pl.loop(0, NCH - 1)
        def _(c):
            jn = lax.rem(c + 1, NBUF)
            for j in range(NBUF):
                @pl.when(jn == j)
                def _():
                    fetch(c + 1, j)
            jc = lax.rem(c, NBUF)
            for j in range(NBUF):
                @pl.when(jc == j)
                def _():
                    @pl.when(c >= NBUF)
                    def _():
                        drain_out(j)
                    consume(c, j)

        cl = NCH - 1
        jl = cl % NBUF
        drain_out(jl)
        consume(cl, jl)

        for j in range(NBUF):
            drain_out(j)

    return gather_k(emb_half, src, dst)


def kernel(edge_attr, edge_index, node_attr, W, b):
    edge_index = edge_index.astype(jnp.int32)
    src = edge_index[0]
    dst = edge_index[1]
    psum, pcnt = _phase1_scatter(edge_attr, src, node_attr.shape[0])
    emb_half = _phase2_node_update(
        psum, pcnt.T, node_attr, W.T, b.reshape(1, -1).astype(jnp.float32)
    )
    return _phase3_gather(emb_half, src, dst)


# speedup vs baseline: 1.0835x; 1.0835x over previous
import functools

import jax
import jax.numpy as jnp
from jax import lax
from jax.experimental import pallas as pl
from jax.experimental.pallas import tpu as pltpu
from jax.experimental.pallas import tpu_sc as plsc

NC = 2
NS = 16
NW = NC * NS
LANES = 16
K = 80
ZR = 40


def _sc_mesh():
    return plsc.VectorSubcoreMesh(
        core_axis_name="c", subcore_axis_name="s", num_cores=NC, num_subcores=NS
    )


def _phase1_scatter(edge_attr, src, n_nodes):
    E, D = edge_attr.shape
    N = n_nodes
    EPW = E // NW
    NCH = EPW // K
    G = D // LANES
    NZC = N // ZR
    NBUF = 3

    @functools.partial(
        pl.kernel,
        out_type=(
            jax.ShapeDtypeStruct((NC, N, D), jnp.float32),
            jax.ShapeDtypeStruct((NW, N), jnp.float32),
        ),
        mesh=_sc_mesh(),
        compiler_params=pltpu.CompilerParams(needs_layout_passes=False),
        scratch_types=[
            pltpu.VMEM_SHARED((N, D), jnp.float32),
            [pltpu.VMEM((K, D), jnp.float32) for _ in range(NBUF)],
            [pltpu.VMEM((K,), jnp.int32) for _ in range(NBUF)],
            pltpu.VMEM((N,), jnp.float32),
            pltpu.VMEM((ZR, D), jnp.float32),
            [pltpu.SemaphoreType.DMA for _ in range(NBUF)],
            [pltpu.SemaphoreType.DMA for _ in range(NBUF)],
            [pltpu.SemaphoreType.DMA for _ in range(NBUF)],
        ],
    )
    def scatter_k(edge_hbm, src_hbm, psum_hbm, pcnt_hbm,
                  acc_sh, rows_v, idx_v, hist_v, zrow_v, si, sr, ss):
        cid = lax.axis_index("c")
        sid = lax.axis_index("s")
        wid = cid * NS + sid

        @pl.loop(0, ZR)
        def _(r):
            @pl.loop(0, G)
            def _(g):
                zrow_v[r, pl.ds(g * LANES, LANES)] = jnp.zeros((LANES,), jnp.float32)

        @pl.loop(0, N // LANES)
        def _(g):
            hist_v[pl.ds(g * LANES, LANES)] = jnp.zeros((LANES,), jnp.float32)

        @pl.loop(0, NZC)
        def _(t):
            @pl.when(lax.rem(t, NS) == sid)
            def _():
                pltpu.sync_copy(zrow_v, acc_sh.at[pl.ds(t * ZR, ZR)])

        plsc.subcore_barrier()

        e0 = wid * EPW
        ones = jnp.ones((LANES,), jnp.float32)

        def fetch(c, j):
            base = e0 + c * K
            pltpu.async_copy(src_hbm.at[pl.ds(base, K)], idx_v[j], si[j])
            pltpu.async_copy(edge_hbm.at[pl.ds(base, K)], rows_v[j], sr[j])

        def drain_scatter(j):
            pltpu.make_async_copy(rows_v[j], acc_sh.at[idx_v[j]], ss[j]).wait()

        def consume(j):
            pltpu.make_async_copy(src_hbm.at[pl.ds(e0, K)], idx_v[j], si[j]).wait()
            pltpu.make_async_copy(edge_hbm.at[pl.ds(e0, K)], rows_v[j], sr[j]).wait()
            pltpu.async_copy(rows_v[j], acc_sh.at[idx_v[j]], ss[j], add=True)
            for g in range(K // LANES):
                iv = idx_v[j][pl.ds(g * LANES, LANES)]
                plsc.addupdate_scatter(hist_v, [iv], ones)

        fetch(0, 0)

        @pl.loop(0, NCH - 1)
        def _(c):
            jn = lax.rem(c + 1, NBUF)
            for j in range(NBUF):
                @pl.when(jn == j)
                def _():
                    @pl.when(c >= NBUF - 1)
                    def _():
                        drain_scatter(j)
                    fetch(c + 1, j)
            jc = lax.rem(c, NBUF)
            for j in range(NBUF):
                @pl.when(jc == j)
                def _():
                    consume(j)

        consume((NCH - 1) % NBUF)
        for j in range(NBUF):
            drain_scatter(j)

        pltpu.sync_copy(hist_v, pcnt_hbm.at[wid])
        plsc.subcore_barrier()

        @pl.loop(0, NZC)
        def _(t):
            @pl.when(lax.rem(t, NS) == sid)
            def _():
                r0 = t * ZR
                pltpu.sync_copy(acc_sh.at[pl.ds(r0, ZR)], psum_hbm.at[cid, pl.ds(r0, ZR)])

    return scatter_k(edge_attr, src)


def _phase2_node_update(psum, pcnt_t, node_attr, wt, b2):
    N, D = node_attr.shape
    BN = 1000

    def body(ps_ref, pc_ref, na_ref, wt_ref, b_ref, out_ref):
        counts = jnp.sum(pc_ref[...], axis=0)[:, None]
        sums = ps_ref[0, :, :] + ps_ref[1, :, :]
        mean = sums / jnp.maximum(counts, 1.0)
        x = na_ref[...] + 0.5 * mean
        z = jnp.dot(x, wt_ref[...], preferred_element_type=jnp.float32) + b_ref[...]
        out_ref[...] = 0.5 / (1.0 + jnp.exp(-z))

    return pl.pallas_call(
        body,
        out_shape=jax.ShapeDtypeStruct((N, D), jnp.float32),
    )(psum, pcnt_t, node_attr, wt, b2)


def _phase3_gather(emb_half, src, dst):
    N, D = emb_half.shape
    E = src.shape[0]
    EPW = E // NW
    NCH = EPW // K
    G = D // LANES

    NBUF = 2

    @functools.partial(
        pl.kernel,
        out_type=jax.ShapeDtypeStruct((E, D), jnp.float32),
        mesh=_sc_mesh(),
        scratch_types=[
            pltpu.VMEM((EPW,), jnp.int32),
            pltpu.VMEM((EPW,), jnp.int32),
            [pltpu.VMEM((K, D), jnp.float32) for _ in range(NBUF)],
            [pltpu.VMEM((K, D), jnp.float32) for _ in range(NBUF)],
            [pltpu.VMEM((K, D), jnp.float32) for _ in range(NBUF)],
            [pltpu.SemaphoreType.DMA for _ in range(NBUF)],
            [pltpu.SemaphoreType.DMA for _ in range(NBUF)],
            [pltpu.SemaphoreType.DMA for _ in range(NBUF)],
        ],
    )
    def gather_k(emb_hbm, src_hbm, dst_hbm, out_hbm,
                 ia_v, ib_v, ba_v, bb_v, bo_v, sa, sb, so):
        cid = lax.axis_index("c")
        sid = lax.axis_index("s")
        wid = cid * NS + sid
        e0 = wid * EPW

        pltpu.sync_copy(src_hbm.at[pl.ds(e0, EPW)], ia_v)
        pltpu.sync_copy(dst_hbm.at[pl.ds(e0, EPW)], ib_v)

        def fetch(c, j):
            o = c * K
            pltpu.async_copy(emb_hbm.at[ia_v.at[pl.ds(o, K)]], ba_v[j], sa[j])
            pltpu.async_copy(emb_hbm.at[ib_v.at[pl.ds(o, K)]], bb_v[j], sb[j])

        def drain_out(j):
            pltpu.make_async_copy(bo_v[j], out_hbm.at[pl.ds(e0, K)], so[j]).wait()

        def consume(c, j):
            o = c * K
            pltpu.make_async_copy(
                emb_hbm.at[ia_v.at[pl.ds(o, K)]], ba_v[j], sa[j]).wait()
            pltpu.make_async_copy(
                emb_hbm.at[ib_v.at[pl.ds(o, K)]], bb_v[j], sb[j]).wait()

            @plsc.parallel_loop(0, K * G, unroll=8)
            def _(i):
                r = lax.shift_right_logical(i, 3)
                g = lax.shift_left(lax.bitwise_and(i, G - 1), 4)
                bo_v[j][r, pl.ds(g, LANES)] = (
                    ba_v[j][r, pl.ds(g, LANES)] + bb_v[j][r, pl.ds(g, LANES)]
                )

            pltpu.async_copy(bo_v[j], out_hbm.at[pl.ds(e0 + c * K, K)], so[j])

        fetch(0, 0)

        @pl.loop(0, NCH - 1)
        def _(c):
            jn = lax.rem(c + 1, NBUF)
            for j in range(NBUF):
                @pl.when(jn == j)
                def _():
                    fetch(c + 1, j)
            jc = lax.rem(c, NBUF)
            for j in range(NBUF):
                @pl.when(jc == j)
                def _():
                    @pl.when(c >= NBUF)
                    def _():
                        drain_out(j)
                    consume(c, j)

        cl = NCH - 1
        jl = cl % NBUF
        drain_out(jl)
        consume(cl, jl)

        for j in range(NBUF):
            drain_out(j)

    return gather_k(emb_half, src, dst)


def kernel(edge_attr, edge_index, node_attr, W, b):
    edge_index = edge_index.astype(jnp.int32)
    src = edge_index[0]
    dst = edge_index[1]
    psum, pcnt = _phase1_scatter(edge_attr, src, node_attr.shape[0])
    emb_half = _phase2_node_update(
        psum, pcnt, node_attr, W.T, b.reshape(1, -1).astype(jnp.float32)
    )
    return _phase3_gather(emb_half, src, dst)
